# baseline (speedup 1.0000x reference)
import functools

import jax
import jax.numpy as jnp
from jax import lax
from jax.experimental import pallas as pl
from jax.experimental.pallas import tpu as pltpu
from jax.experimental.pallas import tpu_sc as plsc

_B, _F, _K, _D = 16384, 100, 50, 64
_NW = 32
_LANES = _B // _NW
_NBUF = 5


def _make_sc_kernel():
    mesh = plsc.VectorSubcoreMesh(core_axis_name="c", subcore_axis_name="s")

    @functools.partial(
        pl.kernel,
        mesh=mesh,
        out_type=jax.ShapeDtypeStruct((_K, _D, _B), jnp.float32),
        scratch_types=[
            [pltpu.SemaphoreType.DMA] * _NBUF,
        ],
    )
    def sc_copy(in_hbm, out_hbm, sem):
        wid = lax.axis_index("s") * 2 + lax.axis_index("c")
        lane0 = wid * _LANES

        def start_copy(slab, slot):
            pltpu.async_copy(
                in_hbm.at[2 * slab, :, pl.ds(lane0, _LANES)],
                out_hbm.at[slab, :, pl.ds(lane0, _LANES)],
                sem[slot])

        def wait_copy(slot):
            pltpu.make_async_copy(
                in_hbm.at[0, :, pl.ds(lane0, _LANES)],
                out_hbm.at[0, :, pl.ds(lane0, _LANES)],
                sem[slot]).wait()

        for k in range(_NBUF):
            start_copy(k, k)

        @pl.loop(_NBUF, _K, step=_NBUF)
        def _(k0):
            for b in range(_NBUF):
                wait_copy(b)
                start_copy(k0 + b, b)

        for b in range(_NBUF):
            wait_copy(b)

    return sc_copy


_SC_KERNEL = _make_sc_kernel()


def kernel(tensor):
    x_t = jnp.transpose(tensor, (1, 2, 0))
    out_t = _SC_KERNEL(x_t)
    return jnp.transpose(out_t, (2, 0, 1))


# speedup vs baseline: 38.2094x; 38.2094x over previous
import functools

import jax
import jax.numpy as jnp
from jax import lax
from jax.experimental import pallas as pl
from jax.experimental.pallas import tpu as pltpu
from jax.experimental.pallas import tpu_sc as plsc

_B, _F, _K, _D = 16384, 100, 50, 64
_NW = 32
_QL = _B // 4
_NCHUNK = _K
_NBUF = 3


def _make_sc_kernel():
    mesh = plsc.VectorSubcoreMesh(core_axis_name="c", subcore_axis_name="s")

    @functools.partial(
        pl.kernel,
        mesh=mesh,
        out_type=jax.ShapeDtypeStruct((_K, _D, _B), jnp.float32),
        scratch_types=[
            pltpu.VMEM((_NBUF, 8, _QL), jnp.float32),
            [pltpu.SemaphoreType.DMA] * _NBUF,
            [pltpu.SemaphoreType.DMA] * _NBUF,
        ],
    )
    def sc_copy(in_hbm, out_hbm, buf, rsem, wsem):
        wid = lax.axis_index("s") * 2 + lax.axis_index("c")
        base = wid * _NCHUNK

        def _addr(c):
            u = base + c
            slab = u // 32
            rem = u % 32
            tr = rem // 4
            q = rem % 4
            return slab, tr * 8, q * _QL

        def start_read(c, slot):
            slab, r0, l0 = _addr(c)
            pltpu.async_copy(
                in_hbm.at[2 * slab, pl.ds(r0, 8), pl.ds(l0, _QL)],
                buf.at[slot], rsem[slot])

        def start_write(c, slot):
            slab, r0, l0 = _addr(c)
            pltpu.async_copy(
                buf.at[slot],
                out_hbm.at[slab, pl.ds(r0, 8), pl.ds(l0, _QL)],
                wsem[slot])

        def wait_read(slot):
            pltpu.make_async_copy(
                in_hbm.at[0, pl.ds(0, 8), pl.ds(0, _QL)],
                buf.at[slot], rsem[slot]).wait()

        def wait_write(slot):
            pltpu.make_async_copy(
                buf.at[slot],
                out_hbm.at[0, pl.ds(0, 8), pl.ds(0, _QL)],
                wsem[slot]).wait()

        start_read(0, 0)
        for c in (0, 1):
            wait_read(c % _NBUF)
            start_write(c, c % _NBUF)
            start_read(c + 1, (c + 1) % _NBUF)

        @pl.loop(2, _NCHUNK - 3, step=_NBUF)
        def _(c0):
            for b in range(_NBUF):
                c = c0 + b
                slot = (2 + b) % _NBUF
                nslot = (2 + b + 1) % _NBUF
                wait_read(slot)
                wait_write(nslot)
                start_write(c, slot)
                start_read(c + 1, nslot)

        for c in (_NCHUNK - 3, _NCHUNK - 2):
            slot = c % _NBUF
            wait_read(slot)
            wait_write((c + 1) % _NBUF)
            start_write(c, slot)
            start_read(c + 1, (c + 1) % _NBUF)
        c = _NCHUNK - 1
        wait_read(c % _NBUF)
        wait_write((c + 1) % _NBUF)
        start_write(c, c % _NBUF)
        for c in (_NCHUNK - 2, _NCHUNK - 1):
            wait_write(c % _NBUF)

    return sc_copy


_SC_KERNEL = _make_sc_kernel()


def kernel(tensor):
    x_t = jnp.transpose(tensor, (1, 2, 0))
    out_t = _SC_KERNEL(x_t)
    return jnp.transpose(out_t, (2, 0, 1))


# speedup vs baseline: 39.2790x; 1.0280x over previous
import functools

import jax
import jax.numpy as jnp
from jax import lax
from jax.experimental import pallas as pl
from jax.experimental.pallas import tpu as pltpu
from jax.experimental.pallas import tpu_sc as plsc

_B, _F, _K, _D = 16384, 100, 50, 64
_NW = 32
_LANES = _B // _NW
_HALF = _D // 2
_NCHUNK = 2 * _K
_NBUF = 6


def _make_sc_kernel():
    mesh = plsc.VectorSubcoreMesh(core_axis_name="c", subcore_axis_name="s")

    @functools.partial(
        pl.kernel,
        mesh=mesh,
        out_type=jax.ShapeDtypeStruct((_K, _D, _B), jnp.float32),
        scratch_types=[
            pltpu.VMEM((_NBUF, _HALF, _LANES), jnp.float32),
            [pltpu.SemaphoreType.DMA] * _NBUF,
            [pltpu.SemaphoreType.DMA] * _NBUF,
        ],
    )
    def sc_copy(in_hbm, out_hbm, buf, rsem, wsem):
        wid = lax.axis_index("s") * 2 + lax.axis_index("c")
        lane0 = wid * _LANES

        def start_read(k, slot):
            pltpu.async_copy(
                in_hbm.at[2 * (k // 2), pl.ds((k % 2) * _HALF, _HALF),
                          pl.ds(lane0, _LANES)],
                buf.at[slot], rsem[slot])

        def start_write(k, slot):
            pltpu.async_copy(
                buf.at[slot],
                out_hbm.at[k // 2, pl.ds((k % 2) * _HALF, _HALF),
                           pl.ds(lane0, _LANES)],
                wsem[slot])

        def wait_read(slot):
            pltpu.make_async_copy(
                in_hbm.at[0, pl.ds(0, _HALF), pl.ds(lane0, _LANES)],
                buf.at[slot], rsem[slot]).wait()

        def wait_write(slot):
            pltpu.make_async_copy(
                buf.at[slot],
                out_hbm.at[0, pl.ds(0, _HALF), pl.ds(lane0, _LANES)],
                wsem[slot]).wait()

        def body(k, slot, nslot, with_ww=True, with_read=True):
            wait_read(slot)
            if with_ww:
                wait_write(nslot)
            start_write(k, slot)
            if with_read:
                start_read(k + 2, nslot)

        start_read(0, 0)
        start_read(1, 1)
        for k in range(0, 8):
            body(k, k % _NBUF, (k + 2) % _NBUF, with_ww=(k >= 4))

        @pl.loop(8, _NCHUNK - 2, step=_NBUF)
        def _(k0):
            for b in range(_NBUF):
                body(k0 + b, (2 + b) % _NBUF, (4 + b) % _NBUF)

        for k in (_NCHUNK - 2, _NCHUNK - 1):
            body(k, k % _NBUF, (k + 2) % _NBUF, with_read=False)
        for k in range(_NCHUNK - 4, _NCHUNK):
            wait_write(k % _NBUF)

    return sc_copy


_SC_KERNEL = _make_sc_kernel()


def kernel(tensor):
    x_t = jnp.transpose(tensor, (1, 2, 0))
    out_t = _SC_KERNEL(x_t)
    return jnp.transpose(out_t, (2, 0, 1))
